# baseline (speedup 1.0000x reference)
import functools

import jax
import jax.numpy as jnp
from jax import lax
from jax.experimental import pallas as pl
from jax.experimental.pallas import tpu as pltpu
from jax.experimental.pallas import tpu_sc as plsc

_NUM_TASKS = 12
_TOKEN_DIM = 128
_BATCH = 16384
_NC = 2
_NS = 16
_NW = _NC * _NS
_BPW = _BATCH // _NW
_L = 16
_CHUNK = 128
_NCHUNK = _BPW // _CHUNK

_mesh = plsc.VectorSubcoreMesh(core_axis_name="c", subcore_axis_name="s")


@functools.partial(
    pl.kernel,
    out_type=jax.ShapeDtypeStruct((_BATCH, 1, _TOKEN_DIM), jnp.float32),
    mesh=_mesh,
    compiler_params=pltpu.CompilerParams(needs_layout_passes=False),
    scratch_types=[
        pltpu.VMEM((_BPW,), jnp.int32),
        pltpu.VMEM((_BPW,), jnp.int32),
        pltpu.VMEM((10, 3), jnp.int32),
        pltpu.VMEM((_NCHUNK, _CHUNK), jnp.int32),
        pltpu.VMEM_SHARED((_NUM_TASKS, 1, _TOKEN_DIM), jnp.float32),
        pltpu.VMEM((2, _CHUNK, 1, _TOKEN_DIM), jnp.float32),
        pltpu.SemaphoreType.DMA,
        pltpu.SemaphoreType.DMA,
        pltpu.SemaphoreType.DMA,
        pltpu.SemaphoreType.DMA,
        pltpu.SemaphoreType.DMA,
    ],
)
def _task_encoder_sc(tm_hbm, pn_hbm, emb_hbm, lut_hbm, out_hbm,
                     tm_v, pn_v, lut_v, idx_v, emb_v, buf_v,
                     gsem0, gsem1, osem0, osem1, isem):
    wid = lax.axis_index("s") * _NC + lax.axis_index("c")
    base = wid * _BPW

    in_copies = [
        pltpu.async_copy(tm_hbm.at[pl.ds(base, _BPW)], tm_v, isem),
        pltpu.async_copy(pn_hbm.at[pl.ds(base, _BPW)], pn_v, isem),
        pltpu.async_copy(lut_hbm, lut_v, isem),
    ]
    sid = lax.axis_index("s")

    @pl.when(sid == 0)
    def _stage_table():
        pltpu.sync_copy(emb_hbm, emb_v)

    for c in in_copies:
        c.wait()
    plsc.subcore_barrier()

    for i in range(_BPW // _L):
        tm = tm_v[pl.ds(i * _L, _L)]
        pn = pn_v[pl.ds(i * _L, _L)]
        idx_v[i // 8, pl.ds((i % 8) * _L, _L)] = plsc.load_gather(lut_v, [tm, pn])

    gsems = (gsem0, gsem1)
    osems = (osem0, osem1)
    out_copies = [None] * _NCHUNK
    for j in range(_NCHUNK):
        p = j % 2
        if j >= 2:
            out_copies[j - 2].wait()
        gather = pltpu.async_copy(emb_v.at[idx_v.at[j]], buf_v.at[p], gsems[p])
        gather.wait()
        out_copies[j] = pltpu.async_copy(
            buf_v.at[p],
            out_hbm.at[pl.ds(base + j * _CHUNK, _CHUNK)],
            osems[p],
        )
    out_copies[_NCHUNK - 2].wait()
    out_copies[_NCHUNK - 1].wait()


def kernel(target_module, port_name, embedding, lookup):
    return _task_encoder_sc(
        target_module, port_name, embedding[:, None, :], lookup
    )


# speedup vs baseline: 1.0312x; 1.0312x over previous
import functools

import jax
import jax.numpy as jnp
from jax import lax
from jax.experimental import pallas as pl
from jax.experimental.pallas import tpu as pltpu
from jax.experimental.pallas import tpu_sc as plsc

_NUM_TASKS = 12
_TOKEN_DIM = 128
_BATCH = 16384
_NC = 2
_NS = 16
_NW = _NC * _NS
_BPW = _BATCH // _NW
_L = 16
_CHUNK = 128
_NCHUNK = _BPW // _CHUNK

_mesh = plsc.VectorSubcoreMesh(core_axis_name="c", subcore_axis_name="s")


@functools.partial(
    pl.kernel,
    out_type=jax.ShapeDtypeStruct((_BATCH, 1, _TOKEN_DIM), jnp.float32),
    mesh=_mesh,
    compiler_params=pltpu.CompilerParams(needs_layout_passes=False),
    scratch_types=[
        pltpu.VMEM((_BPW,), jnp.int32),
        pltpu.VMEM((_BPW,), jnp.int32),
        pltpu.VMEM((10, 3), jnp.int32),
        pltpu.VMEM((_NCHUNK, _CHUNK), jnp.int32),
        pltpu.VMEM_SHARED((_NUM_TASKS, 1, _TOKEN_DIM), jnp.float32),
        pltpu.VMEM((_NCHUNK, _CHUNK, 1, _TOKEN_DIM), jnp.float32),
        pltpu.SemaphoreType.DMA,
        pltpu.SemaphoreType.DMA,
        pltpu.SemaphoreType.DMA,
    ],
)
def _task_encoder_sc(tm_hbm, pn_hbm, emb_hbm, lut_hbm, out_hbm,
                     tm_v, pn_v, lut_v, idx_v, emb_v, buf_v,
                     gsem, osem, isem):
    wid = lax.axis_index("s") * _NC + lax.axis_index("c")
    base = wid * _BPW

    in_copies = [
        pltpu.async_copy(tm_hbm.at[pl.ds(base, _BPW)], tm_v, isem),
        pltpu.async_copy(pn_hbm.at[pl.ds(base, _BPW)], pn_v, isem),
        pltpu.async_copy(lut_hbm, lut_v, isem),
    ]
    sid = lax.axis_index("s")

    @pl.when(sid == 0)
    def _stage_table():
        pltpu.sync_copy(emb_hbm, emb_v)

    for c in in_copies:
        c.wait()

    for i in range(_BPW // _L):
        tm = tm_v[pl.ds(i * _L, _L)]
        pn = pn_v[pl.ds(i * _L, _L)]
        idx_v[i // 8, pl.ds((i % 8) * _L, _L)] = plsc.load_gather(lut_v, [tm, pn])

    plsc.subcore_barrier()

    gathers = [
        pltpu.async_copy(emb_v.at[idx_v.at[j]], buf_v.at[j], gsem)
        for j in range(_NCHUNK)
    ]
    out_copies = []
    for j in range(_NCHUNK):
        gathers[j].wait()
        out_copies.append(
            pltpu.async_copy(
                buf_v.at[j],
                out_hbm.at[pl.ds(base + j * _CHUNK, _CHUNK)],
                osem,
            )
        )
    for c in out_copies:
        c.wait()


def kernel(target_module, port_name, embedding, lookup):
    return _task_encoder_sc(
        target_module, port_name, embedding[:, None, :], lookup
    )
